# baseline (device time: 32481 ns/iter reference)
import functools

import jax
import jax.numpy as jnp
from jax import lax
from jax.experimental import pallas as pl
from jax.experimental.pallas import tpu as pltpu

N_DEV = 16
N_TOK = 1024
N_EXP = 64
CAP = 12
EXP_PER_DEV = N_EXP // N_DEV
N_SLOT = EXP_PER_DEV * CAP
ROWS_PER_DEV = N_TOK // N_DEV


def _moe_kernel(e2d, x, expert_W):
    n, d = x.shape
    h = expert_W.shape[2]

    def body(
        e_ref, x_ref, w_ref, out_ref,
        ybuf, tok_v, keep_v, tok_s, keep_s,
        copy_sems, send_sem, recv_sem,
    ):
        my = lax.axis_index("i")

        out_ref[:, :] = jnp.zeros((ROWS_PER_DEV, h), jnp.float32)

        barrier_sem = pltpu.get_barrier_semaphore()
        for k in range(1, N_DEV):
            pl.semaphore_signal(
                barrier_sem, inc=1,
                device_id=((my + k) % N_DEV,),
                device_id_type=pl.DeviceIdType.MESH,
            )

        eb = e_ref[:, :]

        iota_e = lax.broadcasted_iota(jnp.int32, (n, N_EXP), 1)
        onehot = (eb == iota_e).astype(jnp.bfloat16)

        iota_r = lax.broadcasted_iota(jnp.int32, (n, n), 0)
        iota_c2 = lax.broadcasted_iota(jnp.int32, (n, n), 1)
        tril = (iota_c2 <= iota_r).astype(jnp.bfloat16)
        cum = lax.dot_general(
            tril, onehot,
            (((1,), (0,)), ((), ())),
            preferred_element_type=jnp.float32,
        )
        pcol = (
            jnp.sum(onehot.astype(jnp.float32) * cum, axis=1, keepdims=True)
            - 1.0
        )
        keepcol = pcol < float(CAP)
        keep_v[:, :] = keepcol.astype(jnp.int32)

        iota_c = lax.broadcasted_iota(jnp.int32, (n, CAP), 1).astype(jnp.float32)
        peq = ((pcol == iota_c) & keepcol).astype(jnp.float32)

        iota_k = lax.broadcasted_iota(jnp.int32, (n, EXP_PER_DEV), 1)
        oh_my = (eb - my * EXP_PER_DEV == iota_k).astype(jnp.float32)

        tvals = (
            lax.broadcasted_iota(jnp.int32, (n, CAP), 0).astype(jnp.float32)
            + 1.0
        )
        myslot = lax.dot_general(
            oh_my, peq * tvals,
            (((0,), (0,)), ((), ())),
            precision=lax.Precision.HIGHEST,
        )
        tok_v[:, :] = (jnp.rint(myslot) - 1.0).astype(jnp.int32)

        tok_copy = pltpu.make_async_copy(tok_v, tok_s, copy_sems.at[0])
        tok_copy.start()
        keep_copy = pltpu.make_async_copy(keep_v, keep_s, copy_sems.at[1])
        keep_copy.start()

        for k in range(EXP_PER_DEV):
            g_k = peq * oh_my[:, k : k + 1]
            xg_k = lax.dot_general(
                g_k, x_ref[:, :],
                (((0,), (0,)), ((), ())),
                precision=lax.Precision.HIGHEST,
            )
            ybuf[k * CAP : (k + 1) * CAP, :] = jnp.dot(xg_k, w_ref[k])

        tok_copy.wait()
        keep_copy.wait()
        pl.semaphore_wait(barrier_sem, N_DEV - 1)

        for j in range(N_SLOT):
            t = tok_s[j // CAP, j % CAP]

            @pl.when(t >= 0)
            def _():
                rdma = pltpu.make_async_remote_copy(
                    src_ref=ybuf.at[pl.ds(j, 1)],
                    dst_ref=out_ref.at[pl.ds(lax.rem(t, ROWS_PER_DEV), 1)],
                    send_sem=send_sem,
                    recv_sem=recv_sem,
                    device_id=(lax.div(t, ROWS_PER_DEV),),
                    device_id_type=pl.DeviceIdType.MESH,
                )
                rdma.start()

        n_sent = lax.fori_loop(
            0, N_SLOT,
            lambda j, s: s + jnp.where(tok_s[j // CAP, j % CAP] >= 0, 1, 0),
            0,
        )
        n_recv = lax.fori_loop(
            0, ROWS_PER_DEV,
            lambda j, s: s + keep_s[my * ROWS_PER_DEV + j, 0],
            0,
        )

        dummy = pltpu.make_async_remote_copy(
            src_ref=ybuf.at[pl.ds(0, 1)],
            dst_ref=out_ref.at[pl.ds(0, 1)],
            send_sem=send_sem,
            recv_sem=recv_sem,
            device_id=(my,),
            device_id_type=pl.DeviceIdType.MESH,
        )
        lax.fori_loop(0, n_recv, lambda j, c: (dummy.wait_recv(), c)[1], 0)
        lax.fori_loop(0, n_sent, lambda j, c: (dummy.wait_send(), c)[1], 0)

        @functools.partial(
            pl.run_scoped, second_barrier=pltpu.SemaphoreType.REGULAR
        )
        def _(second_barrier):
            for k in range(1, N_DEV):
                pl.semaphore_signal(
                    second_barrier, inc=1,
                    device_id=((my + k) % N_DEV,),
                    device_id_type=pl.DeviceIdType.MESH,
                )
            pl.semaphore_wait(second_barrier, N_DEV - 1)

    return pl.pallas_call(
        body,
        out_shape=jax.ShapeDtypeStruct((ROWS_PER_DEV, h), jnp.float32),
        in_specs=[
            pl.BlockSpec(memory_space=pltpu.VMEM),
            pl.BlockSpec(memory_space=pltpu.VMEM),
            pl.BlockSpec(memory_space=pltpu.VMEM),
        ],
        out_specs=pl.BlockSpec(memory_space=pltpu.VMEM),
        scratch_shapes=[
            pltpu.VMEM((N_SLOT, h), jnp.float32),
            pltpu.VMEM((EXP_PER_DEV, CAP), jnp.int32),
            pltpu.VMEM((N_TOK, 1), jnp.int32),
            pltpu.SMEM((EXP_PER_DEV, CAP), jnp.int32),
            pltpu.SMEM((N_TOK, 1), jnp.int32),
            pltpu.SemaphoreType.DMA((2,)),
            pltpu.SemaphoreType.DMA,
            pltpu.SemaphoreType.DMA,
        ],
        compiler_params=pltpu.CompilerParams(collective_id=0),
    )(e2d, x, expert_W)


def kernel(x, router_W, route_idx, expert_W):
    del router_W
    return _moe_kernel(route_idx.astype(jnp.int32), x, expert_W)
